# Initial kernel scaffold; baseline (speedup 1.0000x reference)
import jax
import jax.numpy as jnp
from jax.experimental import pallas as pl


def kernel(z, pos, batch, edge_index, emb_table, atom_W, atom_b, edge_emb_W, edge_emb_b, le_W0, le_b0, le_Wh, le_bh, ln_W, ln_b, out_W, out_b):
    raise NotImplementedError("write your pallas kernel here")


# speedup vs baseline: 2.3922x; 2.3922x over previous
import functools

import jax
import jax.numpy as jnp
from jax import lax
from jax.experimental import pallas as pl
from jax.experimental.pallas import tpu as pltpu
from jax.experimental.pallas import tpu_sc as plsc

NN = 10000
NE = 160000
EMB = 128
BINS = 32
MAX_D = 5.0
NL = 5
NG = 128

NODE_BLK = 2000
EDGE_BLK = 640
CH = 128
N_CHUNKS = NE // CH
NC, NS = 2, 16
NW = NC * NS
PER_TILE = -(-N_CHUNKS // NW)
SUB_ROWS = 624


def _silu(x):
    return x * jax.nn.sigmoid(x)


def _make_gather2(d):
    mesh = plsc.VectorSubcoreMesh(core_axis_name="c", subcore_axis_name="s")

    @functools.partial(
        pl.kernel,
        mesh=mesh,
        out_type=[jax.ShapeDtypeStruct((NE, d), jnp.float32),
                  jax.ShapeDtypeStruct((NE, d), jnp.float32)],
        scratch_types=[
            pltpu.VMEM((CH,), jnp.int32),
            pltpu.VMEM((CH,), jnp.int32),
            pltpu.VMEM((CH, d), jnp.float32),
            pltpu.VMEM((CH, d), jnp.float32),
            pltpu.SemaphoreType.DMA,
            pltpu.SemaphoreType.DMA,
        ],
    )
    def gk(table, ridx, cidx, out_r, out_c, idx_r, idx_c, buf_r, buf_c,
           sem_r, sem_c):
        wid = lax.axis_index("s") * NC + lax.axis_index("c")

        def body(i, carry):
            chunk = i * NW + wid

            @pl.when(chunk < N_CHUNKS)
            def _():
                pltpu.sync_copy(ridx.at[chunk], idx_r)
                pltpu.sync_copy(cidx.at[chunk], idx_c)
                cp_r = pltpu.async_copy(table.at[idx_r], buf_r, sem_r)
                cp_c = pltpu.async_copy(table.at[idx_c], buf_c, sem_c)
                cp_r.wait()
                cp_c.wait()
                base = chunk * CH
                pltpu.sync_copy(buf_r, out_r.at[pl.ds(base, CH)])
                pltpu.sync_copy(buf_c, out_c.at[pl.ds(base, CH)])

            return carry

        lax.fori_loop(0, PER_TILE, body, 0)

    return gk


_sc_cache = {}


def _gather2(d, table, ridx, cidx):
    fn = _sc_cache.get(("g", d))
    if fn is None:
        fn = _sc_cache[("g", d)] = _make_gather2(d)
    return fn(table, ridx, cidx)


def _make_scatter_add():
    mesh = plsc.VectorSubcoreMesh(core_axis_name="c", subcore_axis_name="s")

    @functools.partial(
        pl.kernel,
        mesh=mesh,
        out_type=jax.ShapeDtypeStruct((NC, NN, EMB), jnp.float32),
        scratch_types=[
            pltpu.VMEM((CH,), jnp.int32),
            pltpu.VMEM((CH, EMB), jnp.float32),
            pltpu.VMEM_SHARED((NN, EMB), jnp.float32),
        ],
    )
    def sk(vals, cidx, zeros, out, idx_v, buf, acc):
        c = lax.axis_index("c")
        s = lax.axis_index("s")
        wid = s * NC + c
        row0 = s * SUB_ROWS
        pltpu.sync_copy(zeros.at[pl.ds(row0, SUB_ROWS)],
                        acc.at[pl.ds(row0, SUB_ROWS)])

        @pl.when(s == NS - 1)
        def _():
            tail = NS * SUB_ROWS
            pltpu.sync_copy(zeros.at[pl.ds(tail, NN - NS * SUB_ROWS)],
                            acc.at[pl.ds(tail, NN - NS * SUB_ROWS)])

        plsc.subcore_barrier()

        def body(i, carry):
            chunk = i * NW + wid

            @pl.when(chunk < N_CHUNKS)
            def _():
                pltpu.sync_copy(cidx.at[chunk], idx_v)
                pltpu.sync_copy(vals.at[pl.ds(chunk * CH, CH)], buf)
                pltpu.sync_copy(buf, acc.at[idx_v], add=True)

            return carry

        lax.fori_loop(0, PER_TILE, body, 0)
        plsc.subcore_barrier()
        pltpu.sync_copy(acc.at[pl.ds(row0, SUB_ROWS)],
                        out.at[c, pl.ds(row0, SUB_ROWS)])

        @pl.when(s == NS - 1)
        def _():
            tail = NS * SUB_ROWS
            pltpu.sync_copy(acc.at[pl.ds(tail, NN - NS * SUB_ROWS)],
                            out.at[c, pl.ds(tail, NN - NS * SUB_ROWS)])

    return sk


def _scatter_add(vals, cidx, zeros):
    fn = _sc_cache.get("s")
    if fn is None:
        fn = _sc_cache["s"] = _make_scatter_add()
    return fn(vals, cidx, zeros)


def _node_init_body(z_ref, emb_ref, w_ref, b_ref, out_ref):
    zb = z_ref[0, 0, :]
    oh = (zb[:, None] == lax.broadcasted_iota(jnp.int32, (NODE_BLK, 128), 1))
    h = oh.astype(jnp.float32) @ emb_ref[...]
    out_ref[...] = h @ w_ref[...] + b_ref[0:1, :]


def _edge_init_body(pr_ref, pc_ref, ew_ref, eb_ref, out_ref):
    diff = pr_ref[...] - pc_ref[...]
    d2 = jnp.sum(diff * diff, axis=1, keepdims=True)
    d = jnp.sqrt(d2 + 1e-12)
    sigma = MAX_D / (BINS - 1)
    centers = lax.broadcasted_iota(jnp.int32, (1, BINS), 1).astype(jnp.float32) * sigma
    u = (d - centers) * (1.0 / sigma)
    basis = jnp.exp(-0.5 * u * u)
    out_ref[...] = basis @ ew_ref[...] + eb_ref[0:1, :]


def _edge_mlp_body(he_ref, hr_ref, hc_ref, w0_ref, b0_ref, wh_ref, bh_ref,
                   out_ref):
    he = he_ref[...]
    x = (he @ w0_ref[0:EMB] + hr_ref[...] @ w0_ref[EMB:2 * EMB]
         + hc_ref[...] @ w0_ref[2 * EMB:3 * EMB] + b0_ref[0:1, :])
    x = _silu(x)
    for j in range(4):
        x = _silu(x @ wh_ref[j] + bh_ref[j:j + 1, :])
    out_ref[...] = he + x


def _node_mlp_body(part_ref, h_ref, w_ref, b_ref, out_ref):
    agg = part_ref[0] + part_ref[1]
    out_ref[...] = h_ref[...] + _silu(agg @ w_ref[...] + b_ref[0:1, :])


def _readout_body(b3_ref, h_ref, ow_ref, ob_ref, out_ref, sums_ref, cnts_ref):
    i = pl.program_id(0)

    @pl.when(i == 0)
    def _():
        sums_ref[...] = jnp.zeros((NG, EMB), jnp.float32)
        cnts_ref[...] = jnp.zeros((NG, EMB), jnp.float32)

    bb = b3_ref[0, 0, :]
    oh = (bb[:, None] == lax.broadcasted_iota(jnp.int32, (NODE_BLK, NG), 1))
    oh = oh.astype(jnp.float32)
    dn = (((0,), (0,)), ((), ()))
    sums_ref[...] += lax.dot_general(oh, h_ref[...], dn)
    cnts_ref[...] += lax.dot_general(oh, jnp.ones((NODE_BLK, EMB), jnp.float32), dn)

    @pl.when(i == pl.num_programs(0) - 1)
    def _():
        hg = sums_ref[...] / jnp.maximum(cnts_ref[...], 1.0)
        out_ref[...] = hg @ ow_ref[...] + ob_ref[0:1, :]


def _pad_bias(b):
    return jnp.pad(b.reshape(1, -1), ((0, 7), (0, 0)))


def kernel(z, pos, batch, edge_index, emb_table, atom_W, atom_b, edge_emb_W,
           edge_emb_b, le_W0, le_b0, le_Wh, le_bh, ln_W, ln_b, out_W, out_b):
    f32 = jnp.float32
    n_node_blk = NN // NODE_BLK
    n_edge_blk = NE // EDGE_BLK

    row = edge_index[0].reshape(N_CHUNKS, CH).astype(jnp.int32)
    col = edge_index[1].reshape(N_CHUNKS, CH).astype(jnp.int32)
    z3 = z.astype(jnp.int32).reshape(n_node_blk, 1, NODE_BLK)
    b3 = batch.astype(jnp.int32).reshape(n_node_blk, 1, NODE_BLK)
    emb_p = jnp.pad(emb_table, ((0, 128 - emb_table.shape[0]), (0, 0)))
    pos_p = jnp.pad(pos, ((0, 0), (0, 125)))
    ow_p = jnp.pad(out_W, ((0, 0), (0, 127)))
    ob_p = jnp.pad(out_b.reshape(1, 1), ((0, 7), (0, 127)))
    zeros_nn = jnp.zeros((NN, EMB), f32)

    full = lambda *shape: pl.BlockSpec(shape, lambda i: (0,) * len(shape))

    h_node = pl.pallas_call(
        _node_init_body,
        grid=(n_node_blk,),
        in_specs=[
            pl.BlockSpec((1, 1, NODE_BLK), lambda i: (i, 0, 0)),
            full(128, EMB), full(EMB, EMB), full(8, EMB),
        ],
        out_specs=pl.BlockSpec((NODE_BLK, EMB), lambda i: (i, 0)),
        out_shape=jax.ShapeDtypeStruct((NN, EMB), f32),
    )(z3, emb_p, atom_W, _pad_bias(atom_b))

    pr, pc = _gather2(128, pos_p, row, col)
    h_edge = pl.pallas_call(
        _edge_init_body,
        grid=(n_edge_blk,),
        in_specs=[
            pl.BlockSpec((EDGE_BLK, 128), lambda i: (i, 0)),
            pl.BlockSpec((EDGE_BLK, 128), lambda i: (i, 0)),
            full(BINS, EMB), full(8, EMB),
        ],
        out_specs=pl.BlockSpec((EDGE_BLK, EMB), lambda i: (i, 0)),
        out_shape=jax.ShapeDtypeStruct((NE, EMB), f32),
    )(pr, pc, edge_emb_W, _pad_bias(edge_emb_b))

    for l in range(NL):
        hr, hc = _gather2(EMB, h_node, row, col)
        h_edge = pl.pallas_call(
            _edge_mlp_body,
            grid=(n_edge_blk,),
            in_specs=[
                pl.BlockSpec((EDGE_BLK, EMB), lambda i: (i, 0)),
                pl.BlockSpec((EDGE_BLK, EMB), lambda i: (i, 0)),
                pl.BlockSpec((EDGE_BLK, EMB), lambda i: (i, 0)),
                full(3 * EMB, EMB), full(8, EMB),
                full(4, EMB, EMB), full(8, EMB),
            ],
            out_specs=pl.BlockSpec((EDGE_BLK, EMB), lambda i: (i, 0)),
            out_shape=jax.ShapeDtypeStruct((NE, EMB), f32),
        )(h_edge, hr, hc, le_W0[l], _pad_bias(le_b0[l]), le_Wh[l],
          jnp.pad(le_bh[l], ((0, 4), (0, 0))))

        parts = _scatter_add(h_edge, col, zeros_nn)
        h_node = pl.pallas_call(
            _node_mlp_body,
            grid=(n_node_blk,),
            in_specs=[
                pl.BlockSpec((NC, NODE_BLK, EMB), lambda i: (0, i, 0)),
                pl.BlockSpec((NODE_BLK, EMB), lambda i: (i, 0)),
                full(EMB, EMB), full(8, EMB),
            ],
            out_specs=pl.BlockSpec((NODE_BLK, EMB), lambda i: (i, 0)),
            out_shape=jax.ShapeDtypeStruct((NN, EMB), f32),
        )(parts, h_node, ln_W[l], _pad_bias(ln_b[l]))

    res = pl.pallas_call(
        _readout_body,
        grid=(n_node_blk,),
        in_specs=[
            pl.BlockSpec((1, 1, NODE_BLK), lambda i: (i, 0, 0)),
            pl.BlockSpec((NODE_BLK, EMB), lambda i: (i, 0)),
            full(EMB, 128), full(8, 128),
        ],
        out_specs=pl.BlockSpec((NG, 128), lambda i: (0, 0)),
        out_shape=jax.ShapeDtypeStruct((NG, 128), f32),
        scratch_shapes=[
            pltpu.VMEM((NG, EMB), f32),
            pltpu.VMEM((NG, EMB), f32),
        ],
    )(b3, h_node, ow_p, ob_p)
    return res[:, 0:1]


# speedup vs baseline: 2.8547x; 1.1934x over previous
import functools

import jax
import jax.numpy as jnp
from jax import lax
from jax.experimental import pallas as pl
from jax.experimental.pallas import tpu as pltpu
from jax.experimental.pallas import tpu_sc as plsc

NN = 10000
NE = 160000
EMB = 128
BINS = 32
MAX_D = 5.0
NL = 5
NG = 128

NODE_BLK = 2000
EDGE_BLK = 640
CH = 128
N_CHUNKS = NE // CH
NC, NS = 2, 16
NW = NC * NS
PER_TILE = -(-N_CHUNKS // NW)
SUB_ROWS = 624


def _silu(x):
    return x * jax.nn.sigmoid(x)


NB = 3
N_GROUPS = -(-PER_TILE // NB)
SNB = 2
SN_GROUPS = -(-PER_TILE // SNB)


def _make_gather2(d):
    mesh = plsc.VectorSubcoreMesh(core_axis_name="c", subcore_axis_name="s")

    @functools.partial(
        pl.kernel,
        mesh=mesh,
        out_type=[jax.ShapeDtypeStruct((NE, d), jnp.float32),
                  jax.ShapeDtypeStruct((NE, d), jnp.float32)],
        scratch_types=[
            pltpu.VMEM((PER_TILE, CH), jnp.int32),
            pltpu.VMEM((PER_TILE, CH), jnp.int32),
        ] + [pltpu.VMEM((CH, d), jnp.float32)] * (2 * NB)
          + [pltpu.SemaphoreType.DMA] * (2 * NB),
    )
    def gk(table, ridx, cidx, out_r, out_c, idxr_v, idxc_v,
           br0, br1, br2, bc0, bc1, bc2, g0, g1, g2, w0, w1, w2):
        wid = lax.axis_index("s") * NC + lax.axis_index("c")
        bufr, bufc = [br0, br1, br2], [bc0, bc1, bc2]
        gsem, wsem = [g0, g1, g2], [w0, w1, w2]
        pltpu.sync_copy(ridx.at[wid], idxr_v)
        pltpu.sync_copy(cidx.at[wid], idxc_v)

        def fire(jj, b):
            pltpu.async_copy(table.at[idxr_v.at[jj]], bufr[b], gsem[b])
            pltpu.async_copy(table.at[idxc_v.at[jj]], bufc[b], gsem[b])

        def wait2(sems, b):
            pltpu.make_async_copy(out_r.at[pl.ds(0, CH)], bufr[b],
                                  sems[b]).wait()
            pltpu.make_async_copy(out_c.at[pl.ds(0, CH)], bufc[b],
                                  sems[b]).wait()

        for b in range(NB):
            @pl.when(b * NW + wid < N_CHUNKS)
            def _(b=b):
                fire(b, b)

        def body(g, carry):
            for b in range(NB):
                jj = g * NB + b
                ch = jj * NW + wid

                @pl.when(ch < N_CHUNKS)
                def _(jj=jj, ch=ch, b=b):
                    wait2(gsem, b)
                    base = ch * CH
                    pltpu.async_copy(bufr[b], out_r.at[pl.ds(base, CH)],
                                     wsem[b])
                    pltpu.async_copy(bufc[b], out_c.at[pl.ds(base, CH)],
                                     wsem[b])

                @pl.when(ch + NB * NW < N_CHUNKS)
                def _(jj=jj, b=b):
                    wait2(wsem, b)
                    fire(jj + NB, b)

            return carry

        lax.fori_loop(0, N_GROUPS, body, 0)
        for b in range(NB):
            @pl.when(b * NW + wid < N_CHUNKS)
            def _(b=b):
                wait2(wsem, b)

    return gk


_sc_cache = {}


def _gather2(d, table, ridx, cidx):
    fn = _sc_cache.get(("g", d))
    if fn is None:
        fn = _sc_cache[("g", d)] = _make_gather2(d)
    return fn(table, ridx, cidx)


def _make_scatter_add():
    mesh = plsc.VectorSubcoreMesh(core_axis_name="c", subcore_axis_name="s")

    @functools.partial(
        pl.kernel,
        mesh=mesh,
        out_type=jax.ShapeDtypeStruct((NC, NN, EMB), jnp.float32),
        scratch_types=[
            pltpu.VMEM((PER_TILE, CH), jnp.int32),
            pltpu.VMEM_SHARED((NN, EMB), jnp.float32),
        ] + [pltpu.VMEM((CH, EMB), jnp.float32)] * SNB
          + [pltpu.SemaphoreType.DMA] * (2 * SNB),
    )
    def sk(vals, cidx, zeros, out, idx_v, acc, bu0, bu1,
           r0, r1, s0, s1):
        c = lax.axis_index("c")
        s = lax.axis_index("s")
        wid = s * NC + c
        buf = [bu0, bu1]
        rsem, ssem = [r0, r1], [s0, s1]
        row0 = s * SUB_ROWS
        pltpu.sync_copy(zeros.at[pl.ds(row0, SUB_ROWS)],
                        acc.at[pl.ds(row0, SUB_ROWS)])

        @pl.when(s == NS - 1)
        def _():
            tail = NS * SUB_ROWS
            pltpu.sync_copy(zeros.at[pl.ds(tail, NN - NS * SUB_ROWS)],
                            acc.at[pl.ds(tail, NN - NS * SUB_ROWS)])

        plsc.subcore_barrier()
        pltpu.sync_copy(cidx.at[wid], idx_v)

        def wait1(sems, b):
            pltpu.make_async_copy(vals.at[pl.ds(0, CH)], buf[b],
                                  sems[b]).wait()

        for b in range(SNB):
            @pl.when(b * NW + wid < N_CHUNKS)
            def _(b=b):
                pltpu.async_copy(vals.at[pl.ds((b * NW + wid) * CH, CH)],
                                 buf[b], rsem[b])

        def body(g, carry):
            for b in range(SNB):
                jj = g * SNB + b
                ch = jj * NW + wid

                @pl.when(ch < N_CHUNKS)
                def _(jj=jj, b=b):
                    wait1(rsem, b)
                    pltpu.async_copy(buf[b], acc.at[idx_v.at[jj]],
                                     ssem[b], add=True)

                @pl.when(ch + SNB * NW < N_CHUNKS)
                def _(ch=ch, b=b):
                    wait1(ssem, b)
                    nx = ch + SNB * NW
                    pltpu.async_copy(vals.at[pl.ds(nx * CH, CH)],
                                     buf[b], rsem[b])

            return carry

        lax.fori_loop(0, SN_GROUPS, body, 0)
        for b in range(SNB):
            @pl.when(b * NW + wid < N_CHUNKS)
            def _(b=b):
                wait1(ssem, b)
        plsc.subcore_barrier()
        pltpu.sync_copy(acc.at[pl.ds(row0, SUB_ROWS)],
                        out.at[c, pl.ds(row0, SUB_ROWS)])

        @pl.when(s == NS - 1)
        def _():
            tail = NS * SUB_ROWS
            pltpu.sync_copy(acc.at[pl.ds(tail, NN - NS * SUB_ROWS)],
                            out.at[c, pl.ds(tail, NN - NS * SUB_ROWS)])

    return sk


def _scatter_add(vals, cidx, zeros):
    fn = _sc_cache.get("s")
    if fn is None:
        fn = _sc_cache["s"] = _make_scatter_add()
    return fn(vals, cidx, zeros)


def _node_init_body(z_ref, emb_ref, w_ref, b_ref, out_ref):
    zb = z_ref[0, 0, :]
    oh = (zb[:, None] == lax.broadcasted_iota(jnp.int32, (NODE_BLK, 128), 1))
    h = oh.astype(jnp.float32) @ emb_ref[...]
    out_ref[...] = h @ w_ref[...] + b_ref[0:1, :]


def _edge_init_body(pr_ref, pc_ref, ew_ref, eb_ref, out_ref):
    diff = pr_ref[...] - pc_ref[...]
    d2 = jnp.sum(diff * diff, axis=1, keepdims=True)
    d = jnp.sqrt(d2 + 1e-12)
    sigma = MAX_D / (BINS - 1)
    centers = lax.broadcasted_iota(jnp.int32, (1, BINS), 1).astype(jnp.float32) * sigma
    u = (d - centers) * (1.0 / sigma)
    basis = jnp.exp(-0.5 * u * u)
    out_ref[...] = basis @ ew_ref[...] + eb_ref[0:1, :]


def _edge_mlp_body(he_ref, hr_ref, hc_ref, w0_ref, b0_ref, wh_ref, bh_ref,
                   out_ref):
    he = he_ref[...]
    x = (he @ w0_ref[0:EMB] + hr_ref[...] @ w0_ref[EMB:2 * EMB]
         + hc_ref[...] @ w0_ref[2 * EMB:3 * EMB] + b0_ref[0:1, :])
    x = _silu(x)
    for j in range(4):
        x = _silu(x @ wh_ref[j] + bh_ref[j:j + 1, :])
    out_ref[...] = he + x


def _node_mlp_body(part_ref, h_ref, w_ref, b_ref, out_ref):
    agg = part_ref[0] + part_ref[1]
    out_ref[...] = h_ref[...] + _silu(agg @ w_ref[...] + b_ref[0:1, :])


def _readout_body(b3_ref, h_ref, ow_ref, ob_ref, out_ref, sums_ref, cnts_ref):
    i = pl.program_id(0)

    @pl.when(i == 0)
    def _():
        sums_ref[...] = jnp.zeros((NG, EMB), jnp.float32)
        cnts_ref[...] = jnp.zeros((NG, EMB), jnp.float32)

    bb = b3_ref[0, 0, :]
    oh = (bb[:, None] == lax.broadcasted_iota(jnp.int32, (NODE_BLK, NG), 1))
    oh = oh.astype(jnp.float32)
    dn = (((0,), (0,)), ((), ()))
    sums_ref[...] += lax.dot_general(oh, h_ref[...], dn)
    cnts_ref[...] += lax.dot_general(oh, jnp.ones((NODE_BLK, EMB), jnp.float32), dn)

    @pl.when(i == pl.num_programs(0) - 1)
    def _():
        hg = sums_ref[...] / jnp.maximum(cnts_ref[...], 1.0)
        out_ref[...] = hg @ ow_ref[...] + ob_ref[0:1, :]


def _pad_bias(b):
    return jnp.pad(b.reshape(1, -1), ((0, 7), (0, 0)))


def kernel(z, pos, batch, edge_index, emb_table, atom_W, atom_b, edge_emb_W,
           edge_emb_b, le_W0, le_b0, le_Wh, le_bh, ln_W, ln_b, out_W, out_b):
    f32 = jnp.float32
    n_node_blk = NN // NODE_BLK
    n_edge_blk = NE // EDGE_BLK

    def _tilewise(ix):
        p = jnp.pad(ix.reshape(N_CHUNKS, CH).astype(jnp.int32),
                    ((0, NW * PER_TILE - N_CHUNKS), (0, 0)))
        return p.reshape(PER_TILE, NW, CH).transpose(1, 0, 2)

    row = _tilewise(edge_index[0])
    col = _tilewise(edge_index[1])
    z3 = z.astype(jnp.int32).reshape(n_node_blk, 1, NODE_BLK)
    b3 = batch.astype(jnp.int32).reshape(n_node_blk, 1, NODE_BLK)
    emb_p = jnp.pad(emb_table, ((0, 128 - emb_table.shape[0]), (0, 0)))
    pos_p = jnp.pad(pos, ((0, 0), (0, 125)))
    ow_p = jnp.pad(out_W, ((0, 0), (0, 127)))
    ob_p = jnp.pad(out_b.reshape(1, 1), ((0, 7), (0, 127)))
    zeros_nn = jnp.zeros((NN, EMB), f32)

    full = lambda *shape: pl.BlockSpec(shape, lambda i: (0,) * len(shape))

    h_node = pl.pallas_call(
        _node_init_body,
        grid=(n_node_blk,),
        in_specs=[
            pl.BlockSpec((1, 1, NODE_BLK), lambda i: (i, 0, 0)),
            full(128, EMB), full(EMB, EMB), full(8, EMB),
        ],
        out_specs=pl.BlockSpec((NODE_BLK, EMB), lambda i: (i, 0)),
        out_shape=jax.ShapeDtypeStruct((NN, EMB), f32),
    )(z3, emb_p, atom_W, _pad_bias(atom_b))

    pr, pc = _gather2(128, pos_p, row, col)
    h_edge = pl.pallas_call(
        _edge_init_body,
        grid=(n_edge_blk,),
        in_specs=[
            pl.BlockSpec((EDGE_BLK, 128), lambda i: (i, 0)),
            pl.BlockSpec((EDGE_BLK, 128), lambda i: (i, 0)),
            full(BINS, EMB), full(8, EMB),
        ],
        out_specs=pl.BlockSpec((EDGE_BLK, EMB), lambda i: (i, 0)),
        out_shape=jax.ShapeDtypeStruct((NE, EMB), f32),
    )(pr, pc, edge_emb_W, _pad_bias(edge_emb_b))

    for l in range(NL):
        hr, hc = _gather2(EMB, h_node, row, col)
        h_edge = pl.pallas_call(
            _edge_mlp_body,
            grid=(n_edge_blk,),
            in_specs=[
                pl.BlockSpec((EDGE_BLK, EMB), lambda i: (i, 0)),
                pl.BlockSpec((EDGE_BLK, EMB), lambda i: (i, 0)),
                pl.BlockSpec((EDGE_BLK, EMB), lambda i: (i, 0)),
                full(3 * EMB, EMB), full(8, EMB),
                full(4, EMB, EMB), full(8, EMB),
            ],
            out_specs=pl.BlockSpec((EDGE_BLK, EMB), lambda i: (i, 0)),
            out_shape=jax.ShapeDtypeStruct((NE, EMB), f32),
        )(h_edge, hr, hc, le_W0[l], _pad_bias(le_b0[l]), le_Wh[l],
          jnp.pad(le_bh[l], ((0, 4), (0, 0))))

        parts = _scatter_add(h_edge, col, zeros_nn)
        h_node = pl.pallas_call(
            _node_mlp_body,
            grid=(n_node_blk,),
            in_specs=[
                pl.BlockSpec((NC, NODE_BLK, EMB), lambda i: (0, i, 0)),
                pl.BlockSpec((NODE_BLK, EMB), lambda i: (i, 0)),
                full(EMB, EMB), full(8, EMB),
            ],
            out_specs=pl.BlockSpec((NODE_BLK, EMB), lambda i: (i, 0)),
            out_shape=jax.ShapeDtypeStruct((NN, EMB), f32),
        )(parts, h_node, ln_W[l], _pad_bias(ln_b[l]))

    res = pl.pallas_call(
        _readout_body,
        grid=(n_node_blk,),
        in_specs=[
            pl.BlockSpec((1, 1, NODE_BLK), lambda i: (i, 0, 0)),
            pl.BlockSpec((NODE_BLK, EMB), lambda i: (i, 0)),
            full(EMB, 128), full(8, 128),
        ],
        out_specs=pl.BlockSpec((NG, 128), lambda i: (0, 0)),
        out_shape=jax.ShapeDtypeStruct((NG, 128), f32),
        scratch_shapes=[
            pltpu.VMEM((NG, EMB), f32),
            pltpu.VMEM((NG, EMB), f32),
        ],
    )(b3, h_node, ow_p, ob_p)
    return res[:, 0:1]


# speedup vs baseline: 3.0149x; 1.0561x over previous
import functools

import jax
import jax.numpy as jnp
from jax import lax
from jax.experimental import pallas as pl
from jax.experimental.pallas import tpu as pltpu
from jax.experimental.pallas import tpu_sc as plsc

NN = 10000
NE = 160000
EMB = 128
BINS = 32
MAX_D = 5.0
NL = 5
NG = 128

NODE_BLK = 2000
EDGE_BLK = 640
CH = 128
N_CHUNKS = NE // CH
NC, NS = 2, 16
NW = NC * NS
PER_TILE = -(-N_CHUNKS // NW)
SUB_ROWS = 624


def _silu(x):
    return x * jax.nn.sigmoid(x)


NB = 3
N_GROUPS = -(-PER_TILE // NB)
SNB = 3
SN_GROUPS = -(-PER_TILE // SNB)


def _make_gather2(d, dtype):
    mesh = plsc.VectorSubcoreMesh(core_axis_name="c", subcore_axis_name="s")

    @functools.partial(
        pl.kernel,
        mesh=mesh,
        out_type=[jax.ShapeDtypeStruct((NE, d), dtype),
                  jax.ShapeDtypeStruct((NE, d), dtype)],
        scratch_types=[
            pltpu.VMEM((PER_TILE, CH), jnp.int32),
            pltpu.VMEM((PER_TILE, CH), jnp.int32),
        ] + [pltpu.VMEM((CH, d), dtype)] * (2 * NB)
          + [pltpu.SemaphoreType.DMA] * (2 * NB),
    )
    def gk(table, ridx, cidx, out_r, out_c, idxr_v, idxc_v,
           br0, br1, br2, bc0, bc1, bc2, g0, g1, g2, w0, w1, w2):
        wid = lax.axis_index("s") * NC + lax.axis_index("c")
        bufr, bufc = [br0, br1, br2], [bc0, bc1, bc2]
        gsem, wsem = [g0, g1, g2], [w0, w1, w2]
        pltpu.sync_copy(ridx.at[wid], idxr_v)
        pltpu.sync_copy(cidx.at[wid], idxc_v)

        def fire(jj, b):
            pltpu.async_copy(table.at[idxr_v.at[jj]], bufr[b], gsem[b])
            pltpu.async_copy(table.at[idxc_v.at[jj]], bufc[b], gsem[b])

        def wait2(sems, b):
            pltpu.make_async_copy(out_r.at[pl.ds(0, CH)], bufr[b],
                                  sems[b]).wait()
            pltpu.make_async_copy(out_c.at[pl.ds(0, CH)], bufc[b],
                                  sems[b]).wait()

        for b in range(NB):
            @pl.when(b * NW + wid < N_CHUNKS)
            def _(b=b):
                fire(b, b)

        def body(g, carry):
            for b in range(NB):
                jj = g * NB + b
                ch = jj * NW + wid

                @pl.when(ch < N_CHUNKS)
                def _(jj=jj, ch=ch, b=b):
                    wait2(gsem, b)
                    base = ch * CH
                    pltpu.async_copy(bufr[b], out_r.at[pl.ds(base, CH)],
                                     wsem[b])
                    pltpu.async_copy(bufc[b], out_c.at[pl.ds(base, CH)],
                                     wsem[b])

                @pl.when(ch + NB * NW < N_CHUNKS)
                def _(jj=jj, b=b):
                    wait2(wsem, b)
                    fire(jj + NB, b)

            return carry

        lax.fori_loop(0, N_GROUPS, body, 0)
        for b in range(NB):
            @pl.when(b * NW + wid < N_CHUNKS)
            def _(b=b):
                wait2(wsem, b)

    return gk


_sc_cache = {}


def _gather2(d, table, ridx, cidx):
    key = ("g", d, str(table.dtype))
    fn = _sc_cache.get(key)
    if fn is None:
        fn = _sc_cache[key] = _make_gather2(d, table.dtype)
    return fn(table, ridx, cidx)


def _make_scatter_add():
    mesh = plsc.VectorSubcoreMesh(core_axis_name="c", subcore_axis_name="s")

    @functools.partial(
        pl.kernel,
        mesh=mesh,
        out_type=jax.ShapeDtypeStruct((NC, NN, EMB), jnp.float32),
        scratch_types=[
            pltpu.VMEM_SHARED((NN, EMB), jnp.float32),
        ] + [pltpu.VMEM((CH, EMB), jnp.float32)] * SNB
          + [pltpu.VMEM((CH,), jnp.int32)] * SNB
          + [pltpu.SemaphoreType.DMA] * (2 * SNB),
    )
    def sk(vals, cidx, zeros, out, acc, bu0, bu1, bu2, ix0, ix1, ix2,
           r0, r1, r2, s0, s1, s2):
        c = lax.axis_index("c")
        s = lax.axis_index("s")
        wid = s * NC + c
        buf = [bu0, bu1, bu2]
        idxb = [ix0, ix1, ix2]
        rsem, ssem = [r0, r1, r2], [s0, s1, s2]
        row0 = s * SUB_ROWS
        pltpu.sync_copy(zeros.at[pl.ds(row0, SUB_ROWS)],
                        acc.at[pl.ds(row0, SUB_ROWS)])

        @pl.when(s == NS - 1)
        def _():
            tail = NS * SUB_ROWS
            pltpu.sync_copy(zeros.at[pl.ds(tail, NN - NS * SUB_ROWS)],
                            acc.at[pl.ds(tail, NN - NS * SUB_ROWS)])

        plsc.subcore_barrier()

        def fire_read(jj, ch, b):
            pltpu.async_copy(cidx.at[wid, jj], idxb[b], rsem[b])
            pltpu.async_copy(vals.at[pl.ds(ch * CH, CH)], buf[b], rsem[b])

        def wait_read(b):
            pltpu.make_async_copy(cidx.at[0, 0], idxb[b], rsem[b]).wait()
            pltpu.make_async_copy(vals.at[pl.ds(0, CH)], buf[b],
                                  rsem[b]).wait()

        def wait_sadd(b):
            pltpu.make_async_copy(vals.at[pl.ds(0, CH)], buf[b],
                                  ssem[b]).wait()

        for b in range(SNB):
            @pl.when(b * NW + wid < N_CHUNKS)
            def _(b=b):
                fire_read(b, b * NW + wid, b)

        def body(g, carry):
            for b in range(SNB):
                jj = g * SNB + b
                ch = jj * NW + wid

                @pl.when(ch < N_CHUNKS)
                def _(jj=jj, b=b):
                    wait_read(b)
                    pltpu.async_copy(buf[b], acc.at[idxb[b]],
                                     ssem[b], add=True)

                @pl.when(ch + SNB * NW < N_CHUNKS)
                def _(jj=jj, ch=ch, b=b):
                    wait_sadd(b)
                    fire_read(jj + SNB, ch + SNB * NW, b)

            return carry

        lax.fori_loop(0, SN_GROUPS, body, 0)
        for b in range(SNB):
            @pl.when(b * NW + wid < N_CHUNKS)
            def _(b=b):
                wait_sadd(b)
        plsc.subcore_barrier()
        pltpu.sync_copy(acc.at[pl.ds(row0, SUB_ROWS)],
                        out.at[c, pl.ds(row0, SUB_ROWS)])

        @pl.when(s == NS - 1)
        def _():
            tail = NS * SUB_ROWS
            pltpu.sync_copy(acc.at[pl.ds(tail, NN - NS * SUB_ROWS)],
                            out.at[c, pl.ds(tail, NN - NS * SUB_ROWS)])

    return sk


def _scatter_add(vals, cidx, zeros):
    fn = _sc_cache.get("s")
    if fn is None:
        fn = _sc_cache["s"] = _make_scatter_add()
    return fn(vals, cidx, zeros)


def _node_init_body(z_ref, emb_ref, w_ref, b_ref, out_ref):
    zb = z_ref[0, 0, :]
    oh = (zb[:, None] == lax.broadcasted_iota(jnp.int32, (NODE_BLK, 128), 1))
    h = oh.astype(jnp.float32) @ emb_ref[...]
    out_ref[...] = h @ w_ref[...] + b_ref[0:1, :]


def _edge_basis(pr, pc):
    diff = pr - pc
    d2 = jnp.sum(diff * diff, axis=1, keepdims=True)
    d = jnp.sqrt(d2 + 1e-12)
    sigma = MAX_D / (BINS - 1)
    centers = lax.broadcasted_iota(jnp.int32, (1, BINS), 1).astype(
        jnp.float32) * sigma
    u = (d - centers) * (1.0 / sigma)
    return jnp.exp(-0.5 * u * u)


def _edge_mlp0_body(pr_ref, pc_ref, hr_ref, hc_ref, ew_ref, eb_ref,
                    w0_ref, b0_ref, wh_ref, bh_ref, out_ref):
    basis = _edge_basis(pr_ref[...], pc_ref[...])
    he = basis @ ew_ref[...] + eb_ref[0:1, :]
    x = (he @ w0_ref[0:EMB] + hr_ref[...] @ w0_ref[EMB:2 * EMB]
         + hc_ref[...] @ w0_ref[2 * EMB:3 * EMB] + b0_ref[0:1, :])
    x = _silu(x)
    for j in range(4):
        x = _silu(x @ wh_ref[j] + bh_ref[j:j + 1, :])
    out_ref[...] = he + x


def _edge_mlp_body(he_ref, hr_ref, hc_ref, w0_ref, b0_ref, wh_ref, bh_ref,
                   out_ref):
    he = he_ref[...]
    x = (he @ w0_ref[0:EMB] + hr_ref[...] @ w0_ref[EMB:2 * EMB]
         + hc_ref[...] @ w0_ref[2 * EMB:3 * EMB] + b0_ref[0:1, :])
    x = _silu(x)
    for j in range(4):
        x = _silu(x @ wh_ref[j] + bh_ref[j:j + 1, :])
    out_ref[...] = he + x


def _node_mlp_body(part_ref, h_ref, w_ref, b_ref, out_ref):
    agg = part_ref[0] + part_ref[1]
    out_ref[...] = h_ref[...] + _silu(agg @ w_ref[...] + b_ref[0:1, :])


def _readout_body(b3_ref, h_ref, ow_ref, ob_ref, out_ref, sums_ref, cnts_ref):
    i = pl.program_id(0)

    @pl.when(i == 0)
    def _():
        sums_ref[...] = jnp.zeros((NG, EMB), jnp.float32)
        cnts_ref[...] = jnp.zeros((NG, EMB), jnp.float32)

    bb = b3_ref[0, 0, :]
    oh = (bb[:, None] == lax.broadcasted_iota(jnp.int32, (NODE_BLK, NG), 1))
    oh = oh.astype(jnp.float32)
    dn = (((0,), (0,)), ((), ()))
    sums_ref[...] += lax.dot_general(oh, h_ref[...], dn)
    cnts_ref[...] += lax.dot_general(oh, jnp.ones((NODE_BLK, EMB), jnp.float32), dn)

    @pl.when(i == pl.num_programs(0) - 1)
    def _():
        hg = sums_ref[...] / jnp.maximum(cnts_ref[...], 1.0)
        out_ref[...] = hg @ ow_ref[...] + ob_ref[0:1, :]


def _pad_bias(b):
    return jnp.pad(b.reshape(1, -1), ((0, 7), (0, 0)))


def kernel(z, pos, batch, edge_index, emb_table, atom_W, atom_b, edge_emb_W,
           edge_emb_b, le_W0, le_b0, le_Wh, le_bh, ln_W, ln_b, out_W, out_b):
    f32 = jnp.float32
    n_node_blk = NN // NODE_BLK
    n_edge_blk = NE // EDGE_BLK

    def _tilewise(ix):
        p = jnp.pad(ix.reshape(N_CHUNKS, CH).astype(jnp.int32),
                    ((0, NW * PER_TILE - N_CHUNKS), (0, 0)))
        return p.reshape(PER_TILE, NW, CH).transpose(1, 0, 2)

    row = _tilewise(edge_index[0])
    col = _tilewise(edge_index[1])
    z3 = z.astype(jnp.int32).reshape(n_node_blk, 1, NODE_BLK)
    b3 = batch.astype(jnp.int32).reshape(n_node_blk, 1, NODE_BLK)
    emb_p = jnp.pad(emb_table, ((0, 128 - emb_table.shape[0]), (0, 0)))
    pos_p = jnp.pad(pos, ((0, 0), (0, 125)))
    ow_p = jnp.pad(out_W, ((0, 0), (0, 127)))
    ob_p = jnp.pad(out_b.reshape(1, 1), ((0, 7), (0, 127)))
    zeros_nn = jnp.zeros((NN, EMB), f32)

    full = lambda *shape: pl.BlockSpec(shape, lambda i: (0,) * len(shape))

    h_node = pl.pallas_call(
        _node_init_body,
        grid=(n_node_blk,),
        in_specs=[
            pl.BlockSpec((1, 1, NODE_BLK), lambda i: (i, 0, 0)),
            full(128, EMB), full(EMB, EMB), full(8, EMB),
        ],
        out_specs=pl.BlockSpec((NODE_BLK, EMB), lambda i: (i, 0)),
        out_shape=jax.ShapeDtypeStruct((NN, EMB), f32),
    )(z3, emb_p, atom_W, _pad_bias(atom_b))

    pr, pc = _gather2(128, pos_p, row, col)
    eblk = pl.BlockSpec((EDGE_BLK, EMB), lambda i: (i, 0))
    for l in range(NL):
        hr, hc = _gather2(EMB, h_node, row, col)
        if l == 0:
            h_edge = pl.pallas_call(
                _edge_mlp0_body,
                grid=(n_edge_blk,),
                in_specs=[eblk, eblk, eblk, eblk,
                          full(BINS, EMB), full(8, EMB),
                          full(3 * EMB, EMB), full(8, EMB),
                          full(4, EMB, EMB), full(8, EMB)],
                out_specs=eblk,
                out_shape=jax.ShapeDtypeStruct((NE, EMB), f32),
            )(pr, pc, hr, hc, edge_emb_W, _pad_bias(edge_emb_b),
              le_W0[l], _pad_bias(le_b0[l]), le_Wh[l],
              jnp.pad(le_bh[l], ((0, 4), (0, 0))))
        else:
            h_edge = pl.pallas_call(
                _edge_mlp_body,
                grid=(n_edge_blk,),
                in_specs=[eblk, eblk, eblk,
                          full(3 * EMB, EMB), full(8, EMB),
                          full(4, EMB, EMB), full(8, EMB)],
                out_specs=eblk,
                out_shape=jax.ShapeDtypeStruct((NE, EMB), f32),
            )(h_edge, hr, hc, le_W0[l], _pad_bias(le_b0[l]), le_Wh[l],
              jnp.pad(le_bh[l], ((0, 4), (0, 0))))

        parts = _scatter_add(h_edge, col, zeros_nn)
        h_node = pl.pallas_call(
            _node_mlp_body,
            grid=(n_node_blk,),
            in_specs=[
                pl.BlockSpec((NC, NODE_BLK, EMB), lambda i: (0, i, 0)),
                pl.BlockSpec((NODE_BLK, EMB), lambda i: (i, 0)),
                full(EMB, EMB), full(8, EMB),
            ],
            out_specs=pl.BlockSpec((NODE_BLK, EMB), lambda i: (i, 0)),
            out_shape=jax.ShapeDtypeStruct((NN, EMB), f32),
        )(parts, h_node, ln_W[l], _pad_bias(ln_b[l]))

    res = pl.pallas_call(
        _readout_body,
        grid=(n_node_blk,),
        in_specs=[
            pl.BlockSpec((1, 1, NODE_BLK), lambda i: (i, 0, 0)),
            pl.BlockSpec((NODE_BLK, EMB), lambda i: (i, 0)),
            full(EMB, 128), full(8, 128),
        ],
        out_specs=pl.BlockSpec((NG, 128), lambda i: (0, 0)),
        out_shape=jax.ShapeDtypeStruct((NG, 128), f32),
        scratch_shapes=[
            pltpu.VMEM((NG, EMB), f32),
            pltpu.VMEM((NG, EMB), f32),
        ],
    )(b3, h_node, ow_p, ob_p)
    return res[:, 0:1]


# speedup vs baseline: 3.2741x; 1.0860x over previous
import functools

import jax
import jax.numpy as jnp
from jax import lax
from jax.experimental import pallas as pl
from jax.experimental.pallas import tpu as pltpu
from jax.experimental.pallas import tpu_sc as plsc

NN = 10000
NE = 160000
EMB = 128
BINS = 32
MAX_D = 5.0
NL = 5
NG = 128

NODE_BLK = 2000
EDGE_BLK = 640
CH = 128
NC, NS = 2, 16
NW = NC * NS
SUB_ROWS = 624

HALF = NE // 2
NCH_H = HALF // CH
PT_H = -(-NCH_H // NW)
NB = 3


def _silu(x):
    return x * jax.nn.sigmoid(x)


def _make_gather2(d, dtype):
    mesh = plsc.VectorSubcoreMesh(core_axis_name="c", subcore_axis_name="s")
    n_groups = -(-PT_H // NB)

    @functools.partial(
        pl.kernel,
        mesh=mesh,
        out_type=[jax.ShapeDtypeStruct((HALF, d), dtype),
                  jax.ShapeDtypeStruct((HALF, d), dtype)],
        scratch_types=[
            pltpu.VMEM((PT_H, CH), jnp.int32),
            pltpu.VMEM((PT_H, CH), jnp.int32),
        ] + [pltpu.VMEM((CH, d), dtype)] * (2 * NB)
          + [pltpu.SemaphoreType.DMA] * (2 * NB),
    )
    def gk(table, ridx, cidx, out_r, out_c, idxr_v, idxc_v,
           br0, br1, br2, bc0, bc1, bc2, g0, g1, g2, w0, w1, w2):
        wid = lax.axis_index("s") * NC + lax.axis_index("c")
        bufr, bufc = [br0, br1, br2], [bc0, bc1, bc2]
        gsem, wsem = [g0, g1, g2], [w0, w1, w2]
        pltpu.sync_copy(ridx.at[wid], idxr_v)
        pltpu.sync_copy(cidx.at[wid], idxc_v)

        def fire(jj, b):
            pltpu.async_copy(table.at[idxr_v.at[jj]], bufr[b], gsem[b])
            pltpu.async_copy(table.at[idxc_v.at[jj]], bufc[b], gsem[b])

        def wait2(sems, b):
            pltpu.make_async_copy(out_r.at[pl.ds(0, CH)], bufr[b],
                                  sems[b]).wait()
            pltpu.make_async_copy(out_c.at[pl.ds(0, CH)], bufc[b],
                                  sems[b]).wait()

        for b in range(NB):
            @pl.when(b * NW + wid < NCH_H)
            def _(b=b):
                fire(b, b)

        def body(g, carry):
            for b in range(NB):
                jj = g * NB + b
                ch = jj * NW + wid

                @pl.when(ch < NCH_H)
                def _(jj=jj, ch=ch, b=b):
                    wait2(gsem, b)
                    base = ch * CH
                    pltpu.async_copy(bufr[b], out_r.at[pl.ds(base, CH)],
                                     wsem[b])
                    pltpu.async_copy(bufc[b], out_c.at[pl.ds(base, CH)],
                                     wsem[b])

                @pl.when(ch + NB * NW < NCH_H)
                def _(jj=jj, b=b):
                    wait2(wsem, b)
                    fire(jj + NB, b)

            return carry

        lax.fori_loop(0, n_groups, body, 0)
        for b in range(NB):
            @pl.when(b * NW + wid < NCH_H)
            def _(b=b):
                wait2(wsem, b)

    return gk


_sc_cache = {}


def _gather2(d, table, ridx, cidx):
    key = ("g", d, str(table.dtype))
    fn = _sc_cache.get(key)
    if fn is None:
        fn = _sc_cache[key] = _make_gather2(d, table.dtype)
    return fn(table, ridx, cidx)


def _make_scatter_add():
    mesh = plsc.VectorSubcoreMesh(core_axis_name="c", subcore_axis_name="s")
    n_groups = -(-PT_H // NB)

    @functools.partial(
        pl.kernel,
        mesh=mesh,
        out_type=jax.ShapeDtypeStruct((NC, NN, EMB), jnp.float32),
        scratch_types=[
            pltpu.VMEM_SHARED((NN, EMB), jnp.float32),
        ] + [pltpu.VMEM((CH, EMB), jnp.float32)] * NB
          + [pltpu.VMEM((CH,), jnp.int32)] * NB
          + [pltpu.SemaphoreType.DMA] * (2 * NB),
    )
    def sk(vals, cidx, zeros, out, acc, bu0, bu1, bu2, ix0, ix1, ix2,
           r0, r1, r2, s0, s1, s2):
        c = lax.axis_index("c")
        s = lax.axis_index("s")
        wid = s * NC + c
        buf = [bu0, bu1, bu2]
        idxb = [ix0, ix1, ix2]
        rsem, ssem = [r0, r1, r2], [s0, s1, s2]
        row0 = s * SUB_ROWS
        pltpu.sync_copy(zeros.at[pl.ds(row0, SUB_ROWS)],
                        acc.at[pl.ds(row0, SUB_ROWS)])

        @pl.when(s == NS - 1)
        def _():
            tail = NS * SUB_ROWS
            pltpu.sync_copy(zeros.at[pl.ds(tail, NN - NS * SUB_ROWS)],
                            acc.at[pl.ds(tail, NN - NS * SUB_ROWS)])

        plsc.subcore_barrier()

        def fire_read(jj, ch, b):
            pltpu.async_copy(cidx.at[wid, jj], idxb[b], rsem[b])
            pltpu.async_copy(vals.at[pl.ds(ch * CH, CH)], buf[b], rsem[b])

        def wait_read(b):
            pltpu.make_async_copy(cidx.at[0, 0], idxb[b], rsem[b]).wait()
            pltpu.make_async_copy(vals.at[pl.ds(0, CH)], buf[b],
                                  rsem[b]).wait()

        def wait_sadd(b):
            pltpu.make_async_copy(vals.at[pl.ds(0, CH)], buf[b],
                                  ssem[b]).wait()

        for b in range(NB):
            @pl.when(b * NW + wid < NCH_H)
            def _(b=b):
                fire_read(b, b * NW + wid, b)

        def body(g, carry):
            for b in range(NB):
                jj = g * NB + b
                ch = jj * NW + wid

                @pl.when(ch < NCH_H)
                def _(jj=jj, b=b):
                    wait_read(b)
                    pltpu.async_copy(buf[b], acc.at[idxb[b]],
                                     ssem[b], add=True)

                @pl.when(ch + NB * NW < NCH_H)
                def _(jj=jj, ch=ch, b=b):
                    wait_sadd(b)
                    fire_read(jj + NB, ch + NB * NW, b)

            return carry

        lax.fori_loop(0, n_groups, body, 0)
        for b in range(NB):
            @pl.when(b * NW + wid < NCH_H)
            def _(b=b):
                wait_sadd(b)
        plsc.subcore_barrier()
        pltpu.sync_copy(acc.at[pl.ds(row0, SUB_ROWS)],
                        out.at[c, pl.ds(row0, SUB_ROWS)])

        @pl.when(s == NS - 1)
        def _():
            tail = NS * SUB_ROWS
            pltpu.sync_copy(acc.at[pl.ds(tail, NN - NS * SUB_ROWS)],
                            out.at[c, pl.ds(tail, NN - NS * SUB_ROWS)])

    return sk


def _scatter_add(vals, cidx, zeros):
    fn = _sc_cache.get("s")
    if fn is None:
        fn = _sc_cache["s"] = _make_scatter_add()
    return fn(vals, cidx, zeros)


def _node_init_body(z_ref, emb_ref, w_ref, b_ref, out_ref):
    zb = z_ref[0, 0, :]
    oh = (zb[:, None] == lax.broadcasted_iota(jnp.int32, (NODE_BLK, 128), 1))
    h = oh.astype(jnp.float32) @ emb_ref[...]
    out_ref[...] = h @ w_ref[...] + b_ref[0:1, :]


def _edge_basis(pr, pc):
    diff = pr - pc
    d2 = jnp.sum(diff * diff, axis=1, keepdims=True)
    d = jnp.sqrt(d2 + 1e-12)
    sigma = MAX_D / (BINS - 1)
    centers = lax.broadcasted_iota(jnp.int32, (1, BINS), 1).astype(
        jnp.float32) * sigma
    u = (d - centers) * (1.0 / sigma)
    return jnp.exp(-0.5 * u * u)


def _edge_mlp0_body(pr_ref, pc_ref, hr_ref, hc_ref, ew_ref, eb_ref,
                    w0_ref, b0_ref, wh_ref, bh_ref, out_ref):
    basis = _edge_basis(pr_ref[...], pc_ref[...])
    he = basis @ ew_ref[...] + eb_ref[0:1, :]
    x = (he @ w0_ref[0:EMB] + hr_ref[...] @ w0_ref[EMB:2 * EMB]
         + hc_ref[...] @ w0_ref[2 * EMB:3 * EMB] + b0_ref[0:1, :])
    x = _silu(x)
    for j in range(4):
        x = _silu(x @ wh_ref[j] + bh_ref[j:j + 1, :])
    out_ref[...] = he + x


def _edge_mlp_body(he_ref, hr_ref, hc_ref, w0_ref, b0_ref, wh_ref, bh_ref,
                   out_ref):
    he = he_ref[...]
    x = (he @ w0_ref[0:EMB] + hr_ref[...] @ w0_ref[EMB:2 * EMB]
         + hc_ref[...] @ w0_ref[2 * EMB:3 * EMB] + b0_ref[0:1, :])
    x = _silu(x)
    for j in range(4):
        x = _silu(x @ wh_ref[j] + bh_ref[j:j + 1, :])
    out_ref[...] = he + x


def _node_mlp_body(pa_ref, pb_ref, h_ref, w_ref, b_ref, out_ref):
    agg = pa_ref[0] + pa_ref[1] + pb_ref[0] + pb_ref[1]
    out_ref[...] = h_ref[...] + _silu(agg @ w_ref[...] + b_ref[0:1, :])


def _readout_body(b3_ref, h_ref, ow_ref, ob_ref, out_ref, sums_ref, cnts_ref):
    i = pl.program_id(0)

    @pl.when(i == 0)
    def _():
        sums_ref[...] = jnp.zeros((NG, EMB), jnp.float32)
        cnts_ref[...] = jnp.zeros((NG, EMB), jnp.float32)

    bb = b3_ref[0, 0, :]
    oh = (bb[:, None] == lax.broadcasted_iota(jnp.int32, (NODE_BLK, NG), 1))
    oh = oh.astype(jnp.float32)
    dn = (((0,), (0,)), ((), ()))
    sums_ref[...] += lax.dot_general(oh, h_ref[...], dn)
    cnts_ref[...] += lax.dot_general(oh, jnp.ones((NODE_BLK, EMB), jnp.float32), dn)

    @pl.when(i == pl.num_programs(0) - 1)
    def _():
        hg = sums_ref[...] / jnp.maximum(cnts_ref[...], 1.0)
        out_ref[...] = hg @ ow_ref[...] + ob_ref[0:1, :]


def _pad_bias(b):
    return jnp.pad(b.reshape(1, -1), ((0, 7), (0, 0)))


def kernel(z, pos, batch, edge_index, emb_table, atom_W, atom_b, edge_emb_W,
           edge_emb_b, le_W0, le_b0, le_Wh, le_bh, ln_W, ln_b, out_W, out_b):
    f32 = jnp.float32
    n_node_blk = NN // NODE_BLK
    n_edge_blk = HALF // EDGE_BLK

    def _tilewise(ix, base):
        p = ix.reshape(-1, CH)[base:base + NCH_H].astype(jnp.int32)
        p = jnp.pad(p, ((0, NW * PT_H - NCH_H), (0, 0)))
        return p.reshape(PT_H, NW, CH).transpose(1, 0, 2)

    rowA = _tilewise(edge_index[0], 0)
    rowB = _tilewise(edge_index[0], NCH_H)
    colA = _tilewise(edge_index[1], 0)
    colB = _tilewise(edge_index[1], NCH_H)
    z3 = z.astype(jnp.int32).reshape(n_node_blk, 1, NODE_BLK)
    b3 = batch.astype(jnp.int32).reshape(n_node_blk, 1, NODE_BLK)
    emb_p = jnp.pad(emb_table, ((0, 128 - emb_table.shape[0]), (0, 0)))
    pos_p = jnp.pad(pos, ((0, 0), (0, 125)))
    ow_p = jnp.pad(out_W, ((0, 0), (0, 127)))
    ob_p = jnp.pad(out_b.reshape(1, 1), ((0, 7), (0, 127)))
    zeros_nn = jnp.zeros((NN, EMB), f32)

    full = lambda *shape: pl.BlockSpec(shape, lambda i: (0,) * len(shape))
    eblk = pl.BlockSpec((EDGE_BLK, EMB), lambda i: (i, 0))

    h_node = pl.pallas_call(
        _node_init_body,
        grid=(n_node_blk,),
        in_specs=[
            pl.BlockSpec((1, 1, NODE_BLK), lambda i: (i, 0, 0)),
            full(128, EMB), full(EMB, EMB), full(8, EMB),
        ],
        out_specs=pl.BlockSpec((NODE_BLK, EMB), lambda i: (i, 0)),
        out_shape=jax.ShapeDtypeStruct((NN, EMB), f32),
    )(z3, emb_p, atom_W, _pad_bias(atom_b))

    prA, pcA = _gather2(128, pos_p, rowA, colA)
    prB, pcB = _gather2(128, pos_p, rowB, colB)

    def edge_mlp0(pr, pc, hr, hc):
        return pl.pallas_call(
            _edge_mlp0_body,
            grid=(n_edge_blk,),
            in_specs=[eblk, eblk, eblk, eblk,
                      full(BINS, EMB), full(8, EMB),
                      full(3 * EMB, EMB), full(8, EMB),
                      full(4, EMB, EMB), full(8, EMB)],
            out_specs=eblk,
            out_shape=jax.ShapeDtypeStruct((HALF, EMB), f32),
        )(pr, pc, hr, hc, edge_emb_W, _pad_bias(edge_emb_b),
          le_W0[0], _pad_bias(le_b0[0]), le_Wh[0],
          jnp.pad(le_bh[0], ((0, 4), (0, 0))))

    def edge_mlp(l, he, hr, hc):
        return pl.pallas_call(
            _edge_mlp_body,
            grid=(n_edge_blk,),
            in_specs=[eblk, eblk, eblk,
                      full(3 * EMB, EMB), full(8, EMB),
                      full(4, EMB, EMB), full(8, EMB)],
            out_specs=eblk,
            out_shape=jax.ShapeDtypeStruct((HALF, EMB), f32),
        )(he, hr, hc, le_W0[l], _pad_bias(le_b0[l]), le_Wh[l],
          jnp.pad(le_bh[l], ((0, 4), (0, 0))))

    heA = heB = None
    for l in range(NL):
        hrA, hcA = _gather2(EMB, h_node, rowA, colA)
        hrB, hcB = _gather2(EMB, h_node, rowB, colB)
        if l == 0:
            heA = edge_mlp0(prA, pcA, hrA, hcA)
            heB = edge_mlp0(prB, pcB, hrB, hcB)
        else:
            heA = edge_mlp(l, heA, hrA, hcA)
            heB = edge_mlp(l, heB, hrB, hcB)
        pA = _scatter_add(heA, colA, zeros_nn)
        pB = _scatter_add(heB, colB, zeros_nn)
        h_node = pl.pallas_call(
            _node_mlp_body,
            grid=(n_node_blk,),
            in_specs=[
                pl.BlockSpec((NC, NODE_BLK, EMB), lambda i: (0, i, 0)),
                pl.BlockSpec((NC, NODE_BLK, EMB), lambda i: (0, i, 0)),
                pl.BlockSpec((NODE_BLK, EMB), lambda i: (i, 0)),
                full(EMB, EMB), full(8, EMB),
            ],
            out_specs=pl.BlockSpec((NODE_BLK, EMB), lambda i: (i, 0)),
            out_shape=jax.ShapeDtypeStruct((NN, EMB), f32),
        )(pA, pB, h_node, ln_W[l], _pad_bias(ln_b[l]))

    res = pl.pallas_call(
        _readout_body,
        grid=(n_node_blk,),
        in_specs=[
            pl.BlockSpec((1, 1, NODE_BLK), lambda i: (i, 0, 0)),
            pl.BlockSpec((NODE_BLK, EMB), lambda i: (i, 0)),
            full(EMB, 128), full(8, 128),
        ],
        out_specs=pl.BlockSpec((NG, 128), lambda i: (0, 0)),
        out_shape=jax.ShapeDtypeStruct((NG, 128), f32),
        scratch_shapes=[
            pltpu.VMEM((NG, EMB), f32),
            pltpu.VMEM((NG, EMB), f32),
        ],
    )(b3, h_node, ow_p, ob_p)
    return res[:, 0:1]
